# speedup vs baseline: 4.5276x; 4.5276x over previous
import functools

import jax
import jax.numpy as jnp
from jax.experimental import pallas as pl
from jax.experimental.pallas import tpu as pltpu

TOP_K = 2
NUM_EXPERTS = 64
D_MODEL = 768
N_TOKENS = 32768

BLOCK = 1024


def _fused_body(tokens_ref, wg_ref, logits_ref, sel_ref, ew_ref, load_ref):
    i = pl.program_id(0)
    x = tokens_ref[...]
    w = wg_ref[...]
    logits = jnp.dot(x, w, preferred_element_type=jnp.float32)
    logits_ref[...] = logits

    eidx = jax.lax.broadcasted_iota(jnp.int32, (BLOCK, NUM_EXPERTS), 1)
    m1 = jnp.max(logits, axis=1, keepdims=True)
    i1 = jnp.min(jnp.where(logits == m1, eidx, NUM_EXPERTS), axis=1, keepdims=True)
    masked = jnp.where(eidx == i1, -jnp.inf, logits)
    m2 = jnp.max(masked, axis=1, keepdims=True)
    i2 = jnp.min(jnp.where(masked == m2, eidx, NUM_EXPERTS), axis=1, keepdims=True)

    e = jnp.exp(m2 - m1)
    s = 1.0 / (1.0 + e)
    w1 = s
    w2 = e * s

    sel_ref[...] = jnp.concatenate([i1, i2], axis=1)
    ew = jnp.where(eidx == i1, w1, 0.0) + jnp.where(eidx == i2, w2, 0.0)
    ew_ref[...] = ew

    partial = jnp.sum((ew > 0.0).astype(jnp.float32), axis=0, keepdims=True)

    @pl.when(i == 0)
    def _():
        load_ref[...] = jnp.zeros_like(load_ref)

    load_ref[...] += partial


@jax.jit
def kernel(tokens, w_gate, w_noise):
    del w_noise
    grid = (N_TOKENS // BLOCK,)
    logits, sel, ew, load = pl.pallas_call(
        _fused_body,
        grid=grid,
        in_specs=[
            pl.BlockSpec((BLOCK, D_MODEL), lambda i: (i, 0)),
            pl.BlockSpec((D_MODEL, NUM_EXPERTS), lambda i: (0, 0)),
        ],
        out_specs=[
            pl.BlockSpec((BLOCK, NUM_EXPERTS), lambda i: (i, 0)),
            pl.BlockSpec((BLOCK, TOP_K), lambda i: (i, 0)),
            pl.BlockSpec((BLOCK, NUM_EXPERTS), lambda i: (i, 0)),
            pl.BlockSpec((1, NUM_EXPERTS), lambda i: (0, 0)),
        ],
        out_shape=[
            jax.ShapeDtypeStruct((N_TOKENS, NUM_EXPERTS), jnp.float32),
            jax.ShapeDtypeStruct((N_TOKENS, TOP_K), jnp.int32),
            jax.ShapeDtypeStruct((N_TOKENS, NUM_EXPERTS), jnp.float32),
            jax.ShapeDtypeStruct((1, NUM_EXPERTS), jnp.float32),
        ],
    )(tokens, w_gate)
    return logits, sel, ew, load.reshape(NUM_EXPERTS)
